# baseline (speedup 1.0000x reference)
import functools

import jax
import jax.numpy as jnp
from jax import lax
from jax.experimental import pallas as pl
from jax.experimental.pallas import tpu as pltpu
from jax.experimental.pallas import tpu_sc as plsc

NUM_TOKENS = 200
EMB_DIM = 4096
BATCH = 128

NUM_CORES = 2
NUM_SUBCORES = 16
NUM_WORKERS = NUM_CORES * NUM_SUBCORES

SC_BATCH = 32
TC_BATCH = BATCH - SC_BATCH
SC_BATCHES_PER_WORKER = SC_BATCH // NUM_WORKERS

CHUNK_ROWS = 8
NUM_CHUNKS = NUM_TOKENS // CHUNK_ROWS
NUM_SLOTS = 3

TC_GROUP = 4


@functools.partial(
    pl.kernel,
    mesh=plsc.VectorSubcoreMesh(core_axis_name="c", subcore_axis_name="s"),
    out_type=jax.ShapeDtypeStruct((SC_BATCH, NUM_TOKENS, EMB_DIM), jnp.float32),
    scratch_types=(
        [pltpu.VMEM((CHUNK_ROWS, EMB_DIM), jnp.float32)] * NUM_SLOTS
        + [pltpu.SemaphoreType.DMA] * (2 * NUM_SLOTS)
    ),
)
def _sc_broadcast(table_hbm, out_hbm, *scratch):
    bufs = scratch[:NUM_SLOTS]
    lsems = scratch[NUM_SLOTS : 2 * NUM_SLOTS]
    wsems = scratch[2 * NUM_SLOTS :]

    wid = lax.axis_index("s") * NUM_CORES + lax.axis_index("c")
    batch0 = wid * SC_BATCHES_PER_WORKER

    def start_load(c):
        s = c % NUM_SLOTS
        return pltpu.async_copy(
            table_hbm.at[pl.ds(CHUNK_ROWS * c, CHUNK_ROWS)], bufs[s], lsems[s]
        )

    loads = {c: start_load(c) for c in range(NUM_SLOTS)}
    writes = {}
    for c in range(NUM_CHUNKS):
        s = c % NUM_SLOTS
        loads[c].wait()
        writes[c] = [
            pltpu.async_copy(
                bufs[s],
                out_hbm.at[batch0 + i, pl.ds(CHUNK_ROWS * c, CHUNK_ROWS)],
                wsems[s],
            )
            for i in range(SC_BATCHES_PER_WORKER)
        ]
        n = c + NUM_SLOTS - 1
        if NUM_SLOTS <= n < NUM_CHUNKS:
            for h in writes.pop(n - NUM_SLOTS):
                h.wait()
            loads[n] = start_load(n)
    for c in sorted(writes):
        for h in writes[c]:
            h.wait()


def _tc_body(table_ref, out_ref):
    out_ref[...] = jnp.broadcast_to(
        table_ref[...][None], (TC_GROUP, NUM_TOKENS, EMB_DIM)
    )


_tc_broadcast = pl.pallas_call(
    _tc_body,
    grid=(TC_BATCH // TC_GROUP,),
    in_specs=[
        pl.BlockSpec((NUM_TOKENS, EMB_DIM), lambda i: (0, 0)),
    ],
    out_specs=pl.BlockSpec(
        (TC_GROUP, NUM_TOKENS, EMB_DIM), lambda i: (i, 0, 0)
    ),
    out_shape=jax.ShapeDtypeStruct((TC_BATCH, NUM_TOKENS, EMB_DIM), jnp.float32),
)


def kernel(batch_size, virtual_embeddings):
    scale = (jnp.asarray(batch_size, jnp.int32) - BATCH + 1).astype(
        virtual_embeddings.dtype
    )
    table = virtual_embeddings * scale
    sc_part = _sc_broadcast(table)
    tc_part = _tc_broadcast(table)
    return jnp.concatenate([tc_part, sc_part], axis=0)


# speedup vs baseline: 2.0978x; 2.0978x over previous
import functools

import jax
import jax.numpy as jnp
from jax import lax
from jax.experimental import pallas as pl
from jax.experimental.pallas import tpu as pltpu
from jax.experimental.pallas import tpu_sc as plsc

NUM_TOKENS = 200
EMB_DIM = 4096
BATCH = 128

NUM_CORES = 2
NUM_SUBCORES = 16
NUM_WORKERS = NUM_CORES * NUM_SUBCORES
BATCHES_PER_WORKER = BATCH // NUM_WORKERS


@functools.partial(
    pl.kernel,
    mesh=plsc.VectorSubcoreMesh(core_axis_name="c", subcore_axis_name="s"),
    out_type=jax.ShapeDtypeStruct((BATCH, NUM_TOKENS, EMB_DIM), jnp.float32),
    scratch_types=[
        pltpu.VMEM_SHARED((NUM_TOKENS, EMB_DIM), jnp.float32),
        pltpu.SemaphoreType.DMA,
    ],
)
def _sc_broadcast(table_hbm, out_hbm, shared, wsem):
    sid = lax.axis_index("s")
    wid = sid * NUM_CORES + lax.axis_index("c")
    batch0 = wid * BATCHES_PER_WORKER

    @pl.when(sid == 0)
    def _():
        pltpu.sync_copy(table_hbm, shared)

    plsc.subcore_barrier()

    copies = [
        pltpu.async_copy(shared, out_hbm.at[batch0 + i], wsem)
        for i in range(BATCHES_PER_WORKER)
    ]
    for c in copies:
        c.wait()


def kernel(batch_size, virtual_embeddings):
    scale = (jnp.asarray(batch_size, jnp.int32) - BATCH + 1).astype(
        virtual_embeddings.dtype
    )
    return _sc_broadcast(virtual_embeddings * scale)


# speedup vs baseline: 2.7784x; 1.3245x over previous
import functools

import jax
import jax.numpy as jnp
from jax import lax
from jax.experimental import pallas as pl
from jax.experimental.pallas import tpu as pltpu
from jax.experimental.pallas import tpu_sc as plsc

NUM_TOKENS = 200
EMB_DIM = 4096
BATCH = 128

NUM_CORES = 2
NUM_SUBCORES = 16
NUM_WORKERS = NUM_CORES * NUM_SUBCORES

SC_BATCH = 32
TC_BATCH = BATCH - SC_BATCH
SC_BATCHES_PER_WORKER = SC_BATCH // NUM_WORKERS

TC_GROUP = 4


@functools.partial(
    pl.kernel,
    mesh=plsc.VectorSubcoreMesh(core_axis_name="c", subcore_axis_name="s"),
    out_type=jax.ShapeDtypeStruct((BATCH, NUM_TOKENS, EMB_DIM), jnp.float32),
    scratch_types=[
        pltpu.VMEM_SHARED((NUM_TOKENS, EMB_DIM), jnp.float32),
        pltpu.SemaphoreType.DMA,
    ],
)
def _sc_broadcast(table_hbm, out_hbm, shared, wsem):
    sid = lax.axis_index("s")
    wid = sid * NUM_CORES + lax.axis_index("c")
    batch0 = TC_BATCH + wid * SC_BATCHES_PER_WORKER

    @pl.when(sid == 0)
    def _():
        pltpu.sync_copy(table_hbm, shared)

    plsc.subcore_barrier()

    copies = [
        pltpu.async_copy(shared, out_hbm.at[batch0 + i], wsem)
        for i in range(SC_BATCHES_PER_WORKER)
    ]
    for c in copies:
        c.wait()


def _tc_body(table_ref, buf_ref, out_ref):
    del buf_ref
    out_ref[...] = jnp.broadcast_to(
        table_ref[...][None], (TC_GROUP, NUM_TOKENS, EMB_DIM)
    )


_tc_fill = pl.pallas_call(
    _tc_body,
    grid=(TC_BATCH // TC_GROUP,),
    in_specs=[
        pl.BlockSpec((NUM_TOKENS, EMB_DIM), lambda i: (0, 0)),
        pl.BlockSpec(memory_space=pl.ANY),
    ],
    out_specs=pl.BlockSpec(
        (TC_GROUP, NUM_TOKENS, EMB_DIM), lambda i: (i, 0, 0)
    ),
    out_shape=jax.ShapeDtypeStruct((BATCH, NUM_TOKENS, EMB_DIM), jnp.float32),
    input_output_aliases={1: 0},
)


def kernel(batch_size, virtual_embeddings):
    scale = (jnp.asarray(batch_size, jnp.int32) - BATCH + 1).astype(
        virtual_embeddings.dtype
    )
    table = virtual_embeddings * scale
    sc_out = _sc_broadcast(table)
    return _tc_fill(table, sc_out)


# speedup vs baseline: 2.9513x; 1.0622x over previous
import functools

import jax
import jax.numpy as jnp
from jax import lax
from jax.experimental import pallas as pl
from jax.experimental.pallas import tpu as pltpu
from jax.experimental.pallas import tpu_sc as plsc

NUM_TOKENS = 200
EMB_DIM = 4096
BATCH = 128

NUM_CORES = 2
NUM_SUBCORES = 16
NUM_WORKERS = NUM_CORES * NUM_SUBCORES

SC_BATCH = 16
TC_BATCH = BATCH - SC_BATCH

TC_GROUP = 4


@functools.partial(
    pl.kernel,
    mesh=plsc.VectorSubcoreMesh(core_axis_name="c", subcore_axis_name="s"),
    out_type=jax.ShapeDtypeStruct((BATCH, NUM_TOKENS, EMB_DIM), jnp.float32),
    scratch_types=[
        pltpu.VMEM_SHARED((NUM_TOKENS, EMB_DIM), jnp.float32),
        pltpu.SemaphoreType.DMA,
    ],
)
def _sc_broadcast(table_hbm, out_hbm, shared, wsem):
    del wsem
    sid = lax.axis_index("s")
    wid = sid * NUM_CORES + lax.axis_index("c")

    @pl.when(sid == 0)
    def _():
        pltpu.sync_copy(table_hbm, shared)

    plsc.subcore_barrier()

    @pl.when(wid < SC_BATCH)
    def _():
        pltpu.sync_copy(shared, out_hbm.at[TC_BATCH + wid])


def _tc_body(table_ref, buf_ref, out_ref):
    del buf_ref
    out_ref[...] = jnp.broadcast_to(
        table_ref[...][None], (TC_GROUP, NUM_TOKENS, EMB_DIM)
    )


_tc_fill = pl.pallas_call(
    _tc_body,
    grid=(TC_BATCH // TC_GROUP,),
    in_specs=[
        pl.BlockSpec((NUM_TOKENS, EMB_DIM), lambda i: (0, 0)),
        pl.BlockSpec(memory_space=pl.ANY),
    ],
    out_specs=pl.BlockSpec(
        (TC_GROUP, NUM_TOKENS, EMB_DIM), lambda i: (i, 0, 0)
    ),
    out_shape=jax.ShapeDtypeStruct((BATCH, NUM_TOKENS, EMB_DIM), jnp.float32),
    input_output_aliases={1: 0},
)


def kernel(batch_size, virtual_embeddings):
    scale = (jnp.asarray(batch_size, jnp.int32) - BATCH + 1).astype(
        virtual_embeddings.dtype
    )
    table = virtual_embeddings * scale
    sc_out = _sc_broadcast(table)
    return _tc_fill(table, sc_out)
